# Initial kernel scaffold; baseline (speedup 1.0000x reference)
import jax
import jax.numpy as jnp
from jax.experimental import pallas as pl


def kernel(user_emb, item_emb, adj_values, adj_indices):
    raise NotImplementedError("write your pallas kernel here")


# speedup vs baseline: 6.6572x; 6.6572x over previous
import functools

import jax
import jax.numpy as jnp
from jax import lax
from jax.experimental import pallas as pl
from jax.experimental.pallas import tpu as pltpu
from jax.experimental.pallas import tpu_sc as plsc

USERS = 50000
NODES = 100000
EMB = 32
HALF = NODES // 2
DUMP = HALF
TAB_ROWS = 50176
STRIPE = TAB_ROWS // 16
ZCH = 224
NZCH = STRIPE // ZCH

N_EDGES_IN = 1600000
SUB = 128
NSUB = 4
CHUNK = SUB * NSUB
SC_PER_TILE = 196
N_SC = 16 * SC_PER_TILE
E_PAD = N_SC * CHUNK


def _layer_fn(ego, srcx, dstx, valx, out, tab, srcb, dstb, valb, mb, rows,
              zb, gsem):
    c = lax.axis_index("c")
    s = lax.axis_index("s")
    off = c * HALF

    def zbody(i, _):
        zb[i, pl.ds(0, 16)] = jnp.zeros((16,), jnp.float32)
        zb[i, pl.ds(16, 16)] = jnp.zeros((16,), jnp.float32)
        return 0

    lax.fori_loop(0, ZCH, zbody, 0)
    stripe0 = s * STRIPE
    for k in range(NZCH):
        pltpu.sync_copy(zb, tab.at[pl.ds(stripe0 + k * ZCH, ZCH)])
    plsc.subcore_barrier()

    def chunk_body(i, _):
        g = s * SC_PER_TILE + i
        pltpu.sync_copy(srcx.at[g], srcb)
        pltpu.sync_copy(dstx.at[g], dstb)
        pltpu.sync_copy(valx.at[g], valb)
        gh = [pltpu.async_copy(ego.at[srcb.at[j]], rows.at[j], gsem)
              for j in range(NSUB)]
        for h in gh:
            h.wait()
        for j in range(NSUB):
            def gbody(gi, _):
                b = gi * 16
                d = dstb[j, pl.ds(b, 16)]
                ok = (d >= off) & (d < off + HALF)
                mb[j, pl.ds(b, 16)] = jnp.where(ok, d - off, DUMP)
                vv = valb[j, pl.ds(b, 16)]
                for i in range(16):
                    v = vv[i]
                    rows[j, b + i, pl.ds(0, 16)] = (
                        rows[j, b + i, pl.ds(0, 16)] * v)
                    rows[j, b + i, pl.ds(16, 16)] = (
                        rows[j, b + i, pl.ds(16, 16)] * v)
                return 0

            lax.fori_loop(0, SUB // 16, gbody, 0)
            pltpu.sync_copy(rows.at[j], tab.at[mb.at[j]], add=True)
        return 0

    lax.fori_loop(0, SC_PER_TILE, chunk_body, 0)
    plsc.subcore_barrier()

    wb0 = jnp.minimum(s * STRIPE, HALF - STRIPE)
    for k in range(NZCH):
        pltpu.sync_copy(tab.at[pl.ds(wb0 + k * ZCH, ZCH)], zb)
        pltpu.sync_copy(zb, out.at[pl.ds(off + wb0 + k * ZCH, ZCH)])


def _make_layer():
    mesh = plsc.VectorSubcoreMesh(core_axis_name="c", subcore_axis_name="s")
    return pl.kernel(
        _layer_fn,
        mesh=mesh,
        compiler_params=pltpu.CompilerParams(use_tc_tiling_on_sc=False),
        out_type=jax.ShapeDtypeStruct((NODES, EMB), jnp.float32),
        scratch_types=[
            pltpu.VMEM_SHARED((TAB_ROWS, EMB), jnp.float32),
            pltpu.VMEM((NSUB, SUB), jnp.int32),
            pltpu.VMEM((NSUB, SUB), jnp.int32),
            pltpu.VMEM((NSUB, SUB), jnp.float32),
            pltpu.VMEM((NSUB, SUB), jnp.int32),
            pltpu.VMEM((NSUB, SUB, EMB), jnp.float32),
            pltpu.VMEM((ZCH, EMB), jnp.float32),
            pltpu.SemaphoreType.DMA,
        ],
    )


def kernel(user_emb, item_emb, adj_values, adj_indices):
    ego = jnp.concatenate([user_emb, item_emb], axis=0)
    dst = adj_indices[0]
    src = adj_indices[1]
    pad = E_PAD - N_EDGES_IN
    srcx = jnp.pad(src, (0, pad)).reshape(N_SC, NSUB, SUB)
    dstx = jnp.pad(dst, (0, pad)).reshape(N_SC, NSUB, SUB)
    valx = jnp.pad(adj_values, (0, pad)).reshape(N_SC, NSUB, SUB)
    layer = _make_layer()
    e1 = layer(ego, srcx, dstx, valx)
    e2 = layer(e1, srcx, dstx, valx)
    e3 = layer(e2, srcx, dstx, valx)
    mean = (e1 + e2 + e3) * (1.0 / 3.0)
    return mean[:USERS], mean[USERS:]
